# baseline (speedup 1.0000x reference)
import functools

import jax
import jax.numpy as jnp
from jax import lax
from jax.experimental import pallas as pl
from jax.experimental.pallas import tpu as pltpu
from jax.experimental.pallas import tpu_sc as plsc

_ATOMS = 51
_D = 128
_L = 16
_NC = 2
_NS = 16
_NW = _NC * _NS
_N = 4096 * 200
_PER_W = _N // _NW
_E = 64
_CHUNKS = _PER_W // _E
_GROUPS = _E // _L


def _body(x_hbm, p_hbm, out_hbm,
          x_v, i_b0, i_b1, d_b0, d_b1, r_b0, r_b1, out_b0, out_b1,
          gsem0, gsem1, osem0, osem1):
    wid = lax.axis_index("c") * _NS + lax.axis_index("s")
    base = wid * _PER_W
    pltpu.sync_copy(x_hbm.at[pl.ds(base, _PER_W)], x_v)

    def prepass(ci, i_b, d_b):
        eb = ci * _E
        for g in range(_GROUPS):
            xv = x_v[pl.ds(eb + g * _L, _L)]
            xs = xv * float(_ATOMS)
            f = xs.astype(jnp.int32)
            f = jnp.minimum(jnp.maximum(f, 0), _ATOMS - 1)
            d = xs - f.astype(jnp.float32)
            i_b[pl.ds(g * _L, _L)] = f
            d_b[pl.ds(g * _L, _L)] = d

    def issue_gather(i_b, r_b, gsem):
        pltpu.async_copy(p_hbm.at[i_b], r_b, gsem)

    def wait_gather(i_b, r_b, gsem):
        pltpu.make_async_copy(p_hbm.at[i_b], r_b, gsem).wait()

    def blend(r_b, d_b, out_v):
        for g in range(_GROUPS):
            dg = d_b[pl.ds(g * _L, _L)]
            for l in range(_L):
                e = g * _L + l
                eoff = e * _D
                for k in range(_D // _L):
                    out_v[pl.ds(eoff + k * _L, _L)] = dg

    def issue_out(ci, out_v, osem):
        pltpu.async_copy(
            out_v, out_hbm.at[pl.ds((base + ci * _E) * _D, _E * _D)], osem
        )

    def drain_out(ci, out_v, osem):
        pltpu.make_async_copy(
            out_v, out_hbm.at[pl.ds((base + ci * _E) * _D, _E * _D)], osem
        ).wait()

    bufs = ((i_b0, d_b0, r_b0, out_b0, gsem0, osem0),
            (i_b1, d_b1, r_b1, out_b1, gsem1, osem1))

    for b in range(2):
        i_b, d_b, r_b, _, gsem, _ = bufs[b]
        prepass(b, i_b, d_b)
        issue_gather(i_b, r_b, gsem)

    def cbody(i2, _):
        for b in range(2):
            i_b, d_b, r_b, out_v, gsem, osem = bufs[b]
            ci = i2 * 2 + b
            wait_gather(i_b, r_b, gsem)

            @pl.when(i2 > 0)
            def _():
                drain_out(ci - 2, out_v, osem)

            blend(r_b, d_b, out_v)
            issue_out(ci, out_v, osem)

            @pl.when(ci + 2 < _CHUNKS)
            def _():
                prepass(ci + 2, i_b, d_b)
                issue_gather(i_b, r_b, gsem)
        return 0

    lax.fori_loop(0, _CHUNKS // 2, cbody, 0)
    drain_out(_CHUNKS - 2, out_b0, osem0)
    drain_out(_CHUNKS - 1, out_b1, osem1)


@jax.jit
def _run(x_flat, packed):
    mesh = plsc.VectorSubcoreMesh(core_axis_name="c", subcore_axis_name="s")
    k = functools.partial(
        pl.kernel,
        mesh=mesh,
        compiler_params=pltpu.CompilerParams(needs_layout_passes=False),
        out_type=jax.ShapeDtypeStruct((_N * _D,), jnp.float32),
        scratch_types=[
            pltpu.VMEM((_PER_W,), jnp.float32),
            pltpu.VMEM((_E,), jnp.int32),
            pltpu.VMEM((_E,), jnp.int32),
            pltpu.VMEM((_E,), jnp.float32),
            pltpu.VMEM((_E,), jnp.float32),
            pltpu.VMEM((_E, _D), jnp.int32),
            pltpu.VMEM((_E, _D), jnp.int32),
            pltpu.VMEM((_E * _D,), jnp.float32),
            pltpu.VMEM((_E * _D,), jnp.float32),
            pltpu.SemaphoreType.DMA,
            pltpu.SemaphoreType.DMA,
            pltpu.SemaphoreType.DMA,
            pltpu.SemaphoreType.DMA,
        ],
    )(_body)
    return k(x_flat, packed)


def kernel(x, table):
    dt = jnp.concatenate([table[1:] - table[:-1],
                          jnp.zeros((1, _D), table.dtype)])
    hi = lax.bitcast_convert_type(table.astype(jnp.bfloat16),
                                  jnp.uint16).astype(jnp.uint32) << 16
    lo = lax.bitcast_convert_type(dt.astype(jnp.bfloat16),
                                  jnp.uint16).astype(jnp.uint32)
    packed = lax.bitcast_convert_type(hi | lo, jnp.int32)
    out = _run(x.reshape(-1), packed)
    return out.reshape(x.shape[0], x.shape[1], _D)


# speedup vs baseline: 5.0333x; 5.0333x over previous
import functools

import jax
import jax.numpy as jnp
from jax import lax
from jax.experimental import pallas as pl
from jax.experimental.pallas import tpu as pltpu
from jax.experimental.pallas import tpu_sc as plsc

_ATOMS = 51
_D = 128
_L = 16
_NC = 2
_NS = 16
_NW = _NC * _NS
_N = 4096 * 200
_PER_W = _N // _NW
_E = 64
_CHUNKS = _PER_W // _E
_GROUPS = _E // _L


def _body(x_hbm, p_hbm, out_hbm,
          x_v, p_sh, i_b0, i_b1, d_b0, d_b1, r_b0, r_b1, out_b0, out_b1,
          gsem0, gsem1, osem0, osem1):
    sid = lax.axis_index("s")
    wid = lax.axis_index("c") * _NS + sid
    base = wid * _PER_W

    @pl.when(sid == 0)
    def _():
        pltpu.sync_copy(p_hbm, p_sh)

    pltpu.sync_copy(x_hbm.at[pl.ds(base, _PER_W)], x_v)
    plsc.subcore_barrier()

    def prepass(ci, i_b, d_b):
        eb = ci * _E
        for g in range(_GROUPS):
            xv = x_v[pl.ds(eb + g * _L, _L)]
            xs = xv * float(_ATOMS)
            f = xs.astype(jnp.int32)
            f = jnp.minimum(jnp.maximum(f, 0), _ATOMS - 1)
            d = xs - f.astype(jnp.float32)
            i_b[pl.ds(g * _L, _L)] = f
            d_b[pl.ds(g * _L, _L)] = d

    def issue_gather(i_b, r_b, gsem):
        pltpu.async_copy(p_sh.at[i_b], r_b, gsem)

    def wait_gather(i_b, r_b, gsem):
        pltpu.make_async_copy(p_sh.at[i_b], r_b, gsem).wait()

    def blend(r_b, d_b, out_v):
        for g in range(_GROUPS):
            dg = d_b[pl.ds(g * _L, _L)]
            for l in range(_L):
                dv = jnp.broadcast_to(dg[l], (_L,))
                e = g * _L + l
                eoff = e * _D
                gks = [r_b[e, pl.ds(k * _L, _L)] for k in range(_D // _L)]
                for k in range(_D // _L):
                    rf = plsc.bitcast(gks[k], jnp.float32)
                    rd = plsc.bitcast(gks[k] << 16, jnp.float32)
                    out_v[pl.ds(eoff + k * _L, _L)] = rf + dv * rd

    def issue_out(ci, out_v, osem):
        pltpu.async_copy(
            out_v, out_hbm.at[pl.ds((base + ci * _E) * _D, _E * _D)], osem
        )

    def drain_out(ci, out_v, osem):
        pltpu.make_async_copy(
            out_v, out_hbm.at[pl.ds((base + ci * _E) * _D, _E * _D)], osem
        ).wait()

    bufs = ((i_b0, d_b0, r_b0, out_b0, gsem0, osem0),
            (i_b1, d_b1, r_b1, out_b1, gsem1, osem1))

    for b in range(2):
        i_b, d_b, r_b, _, gsem, _ = bufs[b]
        prepass(b, i_b, d_b)
        issue_gather(i_b, r_b, gsem)

    def cbody(i2, _):
        for b in range(2):
            i_b, d_b, r_b, out_v, gsem, osem = bufs[b]
            ci = i2 * 2 + b
            with jax.named_scope("wg"):
                wait_gather(i_b, r_b, gsem)

            @pl.when(i2 > 0)
            def _():
                with jax.named_scope("do"):
                    drain_out(ci - 2, out_v, osem)

            with jax.named_scope("bl"):
                blend(r_b, d_b, out_v)
            issue_out(ci, out_v, osem)

            @pl.when(ci + 2 < _CHUNKS)
            def _():
                with jax.named_scope("pp"):
                    prepass(ci + 2, i_b, d_b)
                    issue_gather(i_b, r_b, gsem)
        return 0

    lax.fori_loop(0, _CHUNKS // 2, cbody, 0)
    drain_out(_CHUNKS - 2, out_b0, osem0)
    drain_out(_CHUNKS - 1, out_b1, osem1)


@jax.jit
def _run(x_flat, packed):
    mesh = plsc.VectorSubcoreMesh(core_axis_name="c", subcore_axis_name="s")
    k = functools.partial(
        pl.kernel,
        mesh=mesh,
        compiler_params=pltpu.CompilerParams(needs_layout_passes=False),
        out_type=jax.ShapeDtypeStruct((_N * _D,), jnp.float32),
        scratch_types=[
            pltpu.VMEM((_PER_W,), jnp.float32),
            pltpu.VMEM_SHARED((_ATOMS, _D), jnp.int32),
            pltpu.VMEM((_E,), jnp.int32),
            pltpu.VMEM((_E,), jnp.int32),
            pltpu.VMEM((_E,), jnp.float32),
            pltpu.VMEM((_E,), jnp.float32),
            pltpu.VMEM((_E, _D), jnp.int32),
            pltpu.VMEM((_E, _D), jnp.int32),
            pltpu.VMEM((_E * _D,), jnp.float32),
            pltpu.VMEM((_E * _D,), jnp.float32),
            pltpu.SemaphoreType.DMA,
            pltpu.SemaphoreType.DMA,
            pltpu.SemaphoreType.DMA,
            pltpu.SemaphoreType.DMA,
        ],
    )(_body)
    return k(x_flat, packed)


def kernel(x, table):
    dt = jnp.concatenate([table[1:] - table[:-1],
                          jnp.zeros((1, _D), table.dtype)])
    hi = lax.bitcast_convert_type(table.astype(jnp.bfloat16),
                                  jnp.uint16).astype(jnp.uint32) << 16
    lo = lax.bitcast_convert_type(dt.astype(jnp.bfloat16),
                                  jnp.uint16).astype(jnp.uint32)
    packed = lax.bitcast_convert_type(hi | lo, jnp.int32)
    out = _run(x.reshape(-1), packed)
    return out.reshape(x.shape[0], x.shape[1], _D)
